# Initial kernel scaffold; baseline (speedup 1.0000x reference)
import jax
import jax.numpy as jnp
from jax.experimental import pallas as pl


def kernel(x, edge_index, W_root0, W_neigh0, W_att0, att_vec0, bn_g0, bn_b0, bn_m0, bn_v0, W_root1, W_neigh1, W_att1, att_vec1, bn_g1, bn_b1, bn_m1, bn_v1, head_W, head_b):
    raise NotImplementedError("write your pallas kernel here")


# speedup vs baseline: 9.7984x; 9.7984x over previous
import functools

import jax
import jax.numpy as jnp
from jax import lax
from jax.experimental import pallas as pl
from jax.experimental.pallas import tpu as pltpu
from jax.experimental.pallas import tpu_sc as plsc

N = 10000
NPAD = 10240
E = 160000
D = 256
C = 128
BN = 400
GRID = N // BN
NT = 32
EPT = E // NT
K = 125
NCHUNK = EPT // K
RPT = NPAD // 16


def _att_body(x_ref, wrT_ref, waT_ref, wnT_ref, avq_ref, avm_ref,
              xr_ref, t0_ref, t1_ref, t2_ref, t3_ref, t4_ref):
    xb = x_ref[...]
    xr_ref[...] = jnp.dot(xb, wrT_ref[...], preferred_element_type=jnp.float32)
    xq = jnp.dot(xb, waT_ref[...], preferred_element_type=jnp.float32)
    xm = jnp.dot(xb, wnT_ref[...], preferred_element_type=jnp.float32)
    s = (jnp.where(xq >= 0, xq, 0.2 * xq) * avq_ref[...]
         + jnp.where(xm >= 0, xm, 0.2 * xm) * avm_ref[...])
    w0 = jnp.exp(jnp.sum(s[:, :D], axis=1, keepdims=True))
    w1 = jnp.exp(jnp.sum(s[:, D:], axis=1, keepdims=True))
    y0 = xm[:, :D] * w0
    y1 = xm[:, D:] * w1
    t0_ref[...] = y0[:, :128]
    t1_ref[...] = y0[:, 128:]
    t2_ref[...] = y1[:, :128]
    t3_ref[...] = y1[:, 128:]
    t4_ref[...] = jnp.concatenate(
        [w0, w1, jnp.zeros((BN, C - 2), jnp.float32)], axis=1)


def _attn_dense(x, wrT, waT, wnT, avq, avm):
    return pl.pallas_call(
        _att_body,
        grid=(GRID,),
        in_specs=[
            pl.BlockSpec((BN, D), lambda i: (i, 0)),
            pl.BlockSpec((D, D), lambda i: (0, 0)),
            pl.BlockSpec((D, 2 * D), lambda i: (0, 0)),
            pl.BlockSpec((D, 2 * D), lambda i: (0, 0)),
            pl.BlockSpec((1, 2 * D), lambda i: (0, 0)),
            pl.BlockSpec((1, 2 * D), lambda i: (0, 0)),
        ],
        out_specs=[pl.BlockSpec((BN, D), lambda i: (i, 0))]
        + [pl.BlockSpec((BN, C), lambda i: (i, 0))] * 5,
        out_shape=[jax.ShapeDtypeStruct((N, D), jnp.float32)]
        + [jax.ShapeDtypeStruct((N, C), jnp.float32)] * 5,
    )(x, wrT, waT, wnT, avq, avm)


_SC_MESH = plsc.VectorSubcoreMesh(core_axis_name="c", subcore_axis_name="s")


@functools.partial(
    pl.kernel,
    out_type=[jax.ShapeDtypeStruct((2, NPAD, C), jnp.float32)] * 5,
    mesh=_SC_MESH,
    scratch_types=[
        pltpu.VMEM((NCHUNK, 1, K), jnp.int32),
        pltpu.VMEM((NCHUNK, 1, K), jnp.int32),
        pltpu.VMEM((K, C), jnp.float32),
        pltpu.VMEM((128, C), jnp.float32),
        pltpu.VMEM_SHARED((NPAD, C), jnp.float32),
        pltpu.SemaphoreType.DMA,
    ],
)
def _sc_push(t0, t1, t2, t3, t4, srcs, dsts, o0, o1, o2, o3, o4,
             sidx, didx, buf, zbuf, acc, sem):
    c = lax.axis_index("c")
    s = lax.axis_index("s")
    wid = c * 16 + s

    pltpu.sync_copy(srcs.at[wid], sidx)
    pltpu.sync_copy(dsts.at[wid], didx)

    z16 = jnp.zeros((16,), jnp.float32)

    def _zrow(i, carry):
        for k in range(C // 16):
            zbuf[i, pl.ds(k * 16, 16)] = z16
        return carry

    lax.fori_loop(0, 128, _zrow, 0)

    for tab, out in ((t0, o0), (t1, o1), (t2, o2), (t3, o3), (t4, o4)):
        for z in range(RPT // 128):
            pltpu.sync_copy(zbuf, acc.at[pl.ds(s * RPT + z * 128, 128)])
        plsc.subcore_barrier()

        def _chunk(j, carry, tab=tab):
            pltpu.async_copy(tab.at[sidx.at[j, 0]], buf, sem).wait()
            pltpu.sync_copy(buf, acc.at[didx.at[j, 0]], add=True)
            return carry

        lax.fori_loop(0, NCHUNK, _chunk, 0)
        plsc.subcore_barrier()

        pltpu.sync_copy(acc.at[pl.ds(s * RPT, RPT)],
                        out.at[c, pl.ds(s * RPT, RPT)])


def _combine(p_ref):
    a = p_ref[...]
    return a[0] + a[1]


def _aggr(xr_ref, p0_ref, p1_ref, p2_ref, p3_ref, p4_ref, sc_ref, bi_ref):
    s0 = _combine(p0_ref)
    s1 = _combine(p1_ref)
    s2 = _combine(p2_ref)
    s3 = _combine(p3_ref)
    s4 = _combine(p4_ref)
    den0 = s4[:, 0:1] + 1e-16
    den1 = s4[:, 1:2] + 1e-16
    a_lo = s0 / den0 + s2 / den1
    a_hi = s1 / den0 + s3 / den1
    h = xr_ref[...] + jnp.concatenate([a_lo, a_hi], axis=1)
    return jnp.maximum(h * sc_ref[...] + bi_ref[...], 0.0)


def _post_body(xr_ref, p0_ref, p1_ref, p2_ref, p3_ref, p4_ref,
               sc_ref, bi_ref, o_ref):
    o_ref[...] = _aggr(xr_ref, p0_ref, p1_ref, p2_ref, p3_ref, p4_ref,
                       sc_ref, bi_ref)


def _post_head_body(xr_ref, p0_ref, p1_ref, p2_ref, p3_ref, p4_ref,
                    sc_ref, bi_ref, hwt_ref, hb_ref, o_ref):
    h = _aggr(xr_ref, p0_ref, p1_ref, p2_ref, p3_ref, p4_ref, sc_ref, bi_ref)
    o_ref[...] = (jnp.dot(h, hwt_ref[...], preferred_element_type=jnp.float32)
                  + hb_ref[...])


_P_SPECS = [
    pl.BlockSpec((BN, D), lambda i: (i, 0)),
    pl.BlockSpec((2, BN, C), lambda i: (0, i, 0)),
    pl.BlockSpec((2, BN, C), lambda i: (0, i, 0)),
    pl.BlockSpec((2, BN, C), lambda i: (0, i, 0)),
    pl.BlockSpec((2, BN, C), lambda i: (0, i, 0)),
    pl.BlockSpec((2, BN, C), lambda i: (0, i, 0)),
    pl.BlockSpec((1, D), lambda i: (0, 0)),
    pl.BlockSpec((1, D), lambda i: (0, 0)),
]


def _post(xr, p0, p1, p2, p3, p4, sc, bi):
    return pl.pallas_call(
        _post_body,
        grid=(GRID,),
        in_specs=_P_SPECS,
        out_specs=pl.BlockSpec((BN, D), lambda i: (i, 0)),
        out_shape=jax.ShapeDtypeStruct((N, D), jnp.float32),
    )(xr, p0, p1, p2, p3, p4, sc, bi)


def _post_head(xr, p0, p1, p2, p3, p4, sc, bi, hwt, hb):
    return pl.pallas_call(
        _post_head_body,
        grid=(GRID,),
        in_specs=_P_SPECS + [
            pl.BlockSpec((D, 128), lambda i: (0, 0)),
            pl.BlockSpec((1, 128), lambda i: (0, 0)),
        ],
        out_specs=pl.BlockSpec((BN, 128), lambda i: (i, 0)),
        out_shape=jax.ShapeDtypeStruct((N, 128), jnp.float32),
    )(xr, p0, p1, p2, p3, p4, sc, bi, hwt, hb)


def kernel(x, edge_index, W_root0, W_neigh0, W_att0, att_vec0,
           bn_g0, bn_b0, bn_m0, bn_v0,
           W_root1, W_neigh1, W_att1, att_vec1,
           bn_g1, bn_b1, bn_m1, bn_v1, head_W, head_b):
    srcs = edge_index[0].reshape(NT, NCHUNK, 1, K)
    dsts = edge_index[1].reshape(NT, NCHUNK, 1, K)

    avq0 = att_vec0[:, :D].reshape(1, 2 * D)
    avm0 = att_vec0[:, D:].reshape(1, 2 * D)
    avq1 = att_vec1[:, :D].reshape(1, 2 * D)
    avm1 = att_vec1[:, D:].reshape(1, 2 * D)

    sc0 = (bn_g0 / jnp.sqrt(bn_v0 + 1e-5)).reshape(1, D)
    bi0 = bn_b0.reshape(1, D) - bn_m0.reshape(1, D) * sc0
    sc1 = (bn_g1 / jnp.sqrt(bn_v1 + 1e-5)).reshape(1, D)
    bi1 = bn_b1.reshape(1, D) - bn_m1.reshape(1, D) * sc1

    hwt = jnp.zeros((D, 128), jnp.float32).at[:, :3].set(head_W.T)
    hb = jnp.zeros((1, 128), jnp.float32).at[0, :3].set(head_b)

    xr1, t10, t11, t12, t13, t14 = _attn_dense(
        x, W_root0.T, W_att0.T, W_neigh0.T, avq0, avm0)
    q1 = _sc_push(t10, t11, t12, t13, t14, srcs, dsts)
    h1 = _post(xr1, *q1, sc0, bi0)

    xr2, t20, t21, t22, t23, t24 = _attn_dense(
        h1, W_root1.T, W_att1.T, W_neigh1.T, avq1, avm1)
    q2 = _sc_push(t20, t21, t22, t23, t24, srcs, dsts)
    out = _post_head(xr2, *q2, sc1, bi1, hwt, hb)
    return out[:, :3]


# speedup vs baseline: 14.2661x; 1.4560x over previous
import functools

import jax
import jax.numpy as jnp
from jax import lax
from jax.experimental import pallas as pl
from jax.experimental.pallas import tpu as pltpu
from jax.experimental.pallas import tpu_sc as plsc

N = 10000
NPAD = 10240
E = 160000
D = 256
C = 128
BN = 400
GRID = N // BN
NT = 32
EPT = E // NT
K = 125
NCHUNK = EPT // K
RPT = NPAD // 16


def _att_body(x_ref, wrT_ref, waT_ref, wnT_ref, avq_ref, avm_ref,
              xr_ref, t0_ref, t1_ref, t2_ref, t3_ref, t4_ref):
    xb = x_ref[...]
    xr_ref[...] = jnp.dot(xb, wrT_ref[...], preferred_element_type=jnp.float32)
    xq = jnp.dot(xb, waT_ref[...], preferred_element_type=jnp.float32)
    xm = jnp.dot(xb, wnT_ref[...], preferred_element_type=jnp.float32)
    s = (jnp.where(xq >= 0, xq, 0.2 * xq) * avq_ref[...]
         + jnp.where(xm >= 0, xm, 0.2 * xm) * avm_ref[...])
    w0 = jnp.exp(jnp.sum(s[:, :D], axis=1, keepdims=True))
    w1 = jnp.exp(jnp.sum(s[:, D:], axis=1, keepdims=True))
    y0 = xm[:, :D] * w0
    y1 = xm[:, D:] * w1
    t0_ref[...] = y0[:, :128]
    t1_ref[...] = y0[:, 128:]
    t2_ref[...] = y1[:, :128]
    t3_ref[...] = y1[:, 128:]
    t4_ref[...] = jnp.concatenate(
        [w0, w1, jnp.zeros((BN, C - 2), jnp.float32)], axis=1)


def _attn_dense(x, wrT, waT, wnT, avq, avm):
    return pl.pallas_call(
        _att_body,
        grid=(GRID,),
        in_specs=[
            pl.BlockSpec((BN, D), lambda i: (i, 0)),
            pl.BlockSpec((D, D), lambda i: (0, 0)),
            pl.BlockSpec((D, 2 * D), lambda i: (0, 0)),
            pl.BlockSpec((D, 2 * D), lambda i: (0, 0)),
            pl.BlockSpec((1, 2 * D), lambda i: (0, 0)),
            pl.BlockSpec((1, 2 * D), lambda i: (0, 0)),
        ],
        out_specs=[pl.BlockSpec((BN, D), lambda i: (i, 0))]
        + [pl.BlockSpec((BN, C), lambda i: (i, 0))] * 5,
        out_shape=[jax.ShapeDtypeStruct((N, D), jnp.float32)]
        + [jax.ShapeDtypeStruct((N, C), jnp.float32)] * 5,
    )(x, wrT, waT, wnT, avq, avm)


_SC_MESH = plsc.VectorSubcoreMesh(core_axis_name="c", subcore_axis_name="s")


@functools.partial(
    pl.kernel,
    out_type=[jax.ShapeDtypeStruct((2, NPAD, C), jnp.float32)] * 5,
    mesh=_SC_MESH,
    scratch_types=[
        pltpu.VMEM((NCHUNK, 1, K), jnp.int32),
        pltpu.VMEM((NCHUNK, 1, K), jnp.int32),
        pltpu.VMEM((K, C), jnp.float32),
        pltpu.VMEM((K, C), jnp.float32),
        pltpu.VMEM((40, C), jnp.float32),
        pltpu.VMEM_SHARED((NPAD, C), jnp.float32),
        pltpu.SemaphoreType.DMA,
        pltpu.SemaphoreType.DMA,
    ],
)
def _sc_push(t0, t1, t2, t3, t4, srcs, dsts, o0, o1, o2, o3, o4,
             sidx, didx, b0, b1, zbuf, acc, g0, g1):
    c = lax.axis_index("c")
    s = lax.axis_index("s")
    wid = c * 16 + s
    bufs = (b0, b1)
    sems = (g0, g1)

    pltpu.sync_copy(srcs.at[wid], sidx)
    pltpu.sync_copy(dsts.at[wid], didx)

    z16 = jnp.zeros((16,), jnp.float32)

    def _zrow(i, carry):
        for k in range(C // 16):
            zbuf[i, pl.ds(k * 16, 16)] = z16
        return carry

    lax.fori_loop(0, 40, _zrow, 0)

    for tab, out in ((t0, o0), (t1, o1), (t2, o2), (t3, o3), (t4, o4)):
        for z in range(RPT // 40):
            pltpu.sync_copy(zbuf, acc.at[pl.ds(s * RPT + z * 40, 40)])
        plsc.subcore_barrier()

        pltpu.async_copy(tab.at[sidx.at[0, 0]], bufs[0], sems[0])

        def _super(g, carry, tab=tab):
            for b in range(2):
                jj = 2 * g + b
                nxt = jj + 1

                @pl.when(nxt < NCHUNK)
                def _():
                    pltpu.async_copy(tab.at[sidx.at[nxt, 0]],
                                     bufs[1 - b], sems[1 - b])

                pltpu.make_async_copy(tab.at[sidx.at[jj, 0]],
                                      bufs[b], sems[b]).wait()
                pltpu.sync_copy(bufs[b], acc.at[didx.at[jj, 0]], add=True)
            return carry

        lax.fori_loop(0, NCHUNK // 2, _super, 0)
        plsc.subcore_barrier()

        pltpu.sync_copy(acc.at[pl.ds(s * RPT, RPT)],
                        out.at[c, pl.ds(s * RPT, RPT)])


def _combine(p_ref):
    a = p_ref[...]
    return a[0] + a[1]


def _aggr(xr_ref, p0_ref, p1_ref, p2_ref, p3_ref, p4_ref, sc_ref, bi_ref):
    s0 = _combine(p0_ref)
    s1 = _combine(p1_ref)
    s2 = _combine(p2_ref)
    s3 = _combine(p3_ref)
    s4 = _combine(p4_ref)
    den0 = s4[:, 0:1] + 1e-16
    den1 = s4[:, 1:2] + 1e-16
    a_lo = s0 / den0 + s2 / den1
    a_hi = s1 / den0 + s3 / den1
    h = xr_ref[...] + jnp.concatenate([a_lo, a_hi], axis=1)
    return jnp.maximum(h * sc_ref[...] + bi_ref[...], 0.0)


def _post_body(xr_ref, p0_ref, p1_ref, p2_ref, p3_ref, p4_ref,
               sc_ref, bi_ref, o_ref):
    o_ref[...] = _aggr(xr_ref, p0_ref, p1_ref, p2_ref, p3_ref, p4_ref,
                       sc_ref, bi_ref)


def _post_head_body(xr_ref, p0_ref, p1_ref, p2_ref, p3_ref, p4_ref,
                    sc_ref, bi_ref, hwt_ref, hb_ref, o_ref):
    h = _aggr(xr_ref, p0_ref, p1_ref, p2_ref, p3_ref, p4_ref, sc_ref, bi_ref)
    o_ref[...] = (jnp.dot(h, hwt_ref[...], preferred_element_type=jnp.float32)
                  + hb_ref[...])


_P_SPECS = [
    pl.BlockSpec((BN, D), lambda i: (i, 0)),
    pl.BlockSpec((2, BN, C), lambda i: (0, i, 0)),
    pl.BlockSpec((2, BN, C), lambda i: (0, i, 0)),
    pl.BlockSpec((2, BN, C), lambda i: (0, i, 0)),
    pl.BlockSpec((2, BN, C), lambda i: (0, i, 0)),
    pl.BlockSpec((2, BN, C), lambda i: (0, i, 0)),
    pl.BlockSpec((1, D), lambda i: (0, 0)),
    pl.BlockSpec((1, D), lambda i: (0, 0)),
]


def _post(xr, p0, p1, p2, p3, p4, sc, bi):
    return pl.pallas_call(
        _post_body,
        grid=(GRID,),
        in_specs=_P_SPECS,
        out_specs=pl.BlockSpec((BN, D), lambda i: (i, 0)),
        out_shape=jax.ShapeDtypeStruct((N, D), jnp.float32),
    )(xr, p0, p1, p2, p3, p4, sc, bi)


def _post_head(xr, p0, p1, p2, p3, p4, sc, bi, hwt, hb):
    return pl.pallas_call(
        _post_head_body,
        grid=(GRID,),
        in_specs=_P_SPECS + [
            pl.BlockSpec((D, 128), lambda i: (0, 0)),
            pl.BlockSpec((1, 128), lambda i: (0, 0)),
        ],
        out_specs=pl.BlockSpec((BN, 128), lambda i: (i, 0)),
        out_shape=jax.ShapeDtypeStruct((N, 128), jnp.float32),
    )(xr, p0, p1, p2, p3, p4, sc, bi, hwt, hb)


def kernel(x, edge_index, W_root0, W_neigh0, W_att0, att_vec0,
           bn_g0, bn_b0, bn_m0, bn_v0,
           W_root1, W_neigh1, W_att1, att_vec1,
           bn_g1, bn_b1, bn_m1, bn_v1, head_W, head_b):
    srcs = edge_index[0].reshape(NT, NCHUNK, 1, K)
    dsts = edge_index[1].reshape(NT, NCHUNK, 1, K)

    avq0 = att_vec0[:, :D].reshape(1, 2 * D)
    avm0 = att_vec0[:, D:].reshape(1, 2 * D)
    avq1 = att_vec1[:, :D].reshape(1, 2 * D)
    avm1 = att_vec1[:, D:].reshape(1, 2 * D)

    sc0 = (bn_g0 / jnp.sqrt(bn_v0 + 1e-5)).reshape(1, D)
    bi0 = bn_b0.reshape(1, D) - bn_m0.reshape(1, D) * sc0
    sc1 = (bn_g1 / jnp.sqrt(bn_v1 + 1e-5)).reshape(1, D)
    bi1 = bn_b1.reshape(1, D) - bn_m1.reshape(1, D) * sc1

    hwt = jnp.zeros((D, 128), jnp.float32).at[:, :3].set(head_W.T)
    hb = jnp.zeros((1, 128), jnp.float32).at[0, :3].set(head_b)

    xr1, t10, t11, t12, t13, t14 = _attn_dense(
        x, W_root0.T, W_att0.T, W_neigh0.T, avq0, avm0)
    q1 = _sc_push(t10, t11, t12, t13, t14, srcs, dsts)
    h1 = _post(xr1, *q1, sc0, bi0)

    xr2, t20, t21, t22, t23, t24 = _attn_dense(
        h1, W_root1.T, W_att1.T, W_neigh1.T, avq1, avm1)
    q2 = _sc_push(t20, t21, t22, t23, t24, srcs, dsts)
    out = _post_head(xr2, *q2, sc1, bi1, hwt, hb)
    return out[:, :3]


# speedup vs baseline: 14.6817x; 1.0291x over previous
import functools

import jax
import jax.numpy as jnp
from jax import lax
from jax.experimental import pallas as pl
from jax.experimental.pallas import tpu as pltpu
from jax.experimental.pallas import tpu_sc as plsc

N = 10000
NPAD = 10240
E = 160000
D = 256
C = 128
BN = 400
GRID = N // BN
NT = 32
EPT = E // NT
K = 125
NCHUNK = EPT // K
RPT = NPAD // 16


def _dense_tables(xb, wrT_ref, waT_ref, wnT_ref, avq_ref, avm_ref,
                  xr_ref, t0_ref, t1_ref, t2_ref, t3_ref, t4_ref):
    xr_ref[...] = jnp.dot(xb, wrT_ref[...], preferred_element_type=jnp.float32)
    xq = jnp.dot(xb, waT_ref[...], preferred_element_type=jnp.float32)
    xm = jnp.dot(xb, wnT_ref[...], preferred_element_type=jnp.float32)
    s = (jnp.where(xq >= 0, xq, 0.2 * xq) * avq_ref[...]
         + jnp.where(xm >= 0, xm, 0.2 * xm) * avm_ref[...])
    w0 = jnp.exp(jnp.sum(s[:, :D], axis=1, keepdims=True))
    w1 = jnp.exp(jnp.sum(s[:, D:], axis=1, keepdims=True))
    y0 = xm[:, :D] * w0
    y1 = xm[:, D:] * w1
    t0_ref[...] = y0[:, :128]
    t1_ref[...] = y0[:, 128:]
    t2_ref[...] = y1[:, :128]
    t3_ref[...] = y1[:, 128:]
    t4_ref[...] = jnp.concatenate(
        [w0, w1, jnp.zeros((BN, C - 2), jnp.float32)], axis=1)


def _att_body(x_ref, wrT_ref, waT_ref, wnT_ref, avq_ref, avm_ref,
              xr_ref, t0_ref, t1_ref, t2_ref, t3_ref, t4_ref):
    _dense_tables(x_ref[...], wrT_ref, waT_ref, wnT_ref, avq_ref, avm_ref,
                  xr_ref, t0_ref, t1_ref, t2_ref, t3_ref, t4_ref)


def _attn_dense(x, wrT, waT, wnT, avq, avm):
    return pl.pallas_call(
        _att_body,
        grid=(GRID,),
        in_specs=[
            pl.BlockSpec((BN, D), lambda i: (i, 0)),
            pl.BlockSpec((D, D), lambda i: (0, 0)),
            pl.BlockSpec((D, 2 * D), lambda i: (0, 0)),
            pl.BlockSpec((D, 2 * D), lambda i: (0, 0)),
            pl.BlockSpec((1, 2 * D), lambda i: (0, 0)),
            pl.BlockSpec((1, 2 * D), lambda i: (0, 0)),
        ],
        out_specs=[pl.BlockSpec((BN, D), lambda i: (i, 0))]
        + [pl.BlockSpec((BN, C), lambda i: (i, 0))] * 5,
        out_shape=[jax.ShapeDtypeStruct((N, D), jnp.float32)]
        + [jax.ShapeDtypeStruct((N, C), jnp.float32)] * 5,
    )(x, wrT, waT, wnT, avq, avm)


_SC_MESH = plsc.VectorSubcoreMesh(core_axis_name="c", subcore_axis_name="s")


@functools.partial(
    pl.kernel,
    out_type=[jax.ShapeDtypeStruct((2, NPAD, C), jnp.float32)] * 5,
    mesh=_SC_MESH,
    scratch_types=[
        pltpu.VMEM((NCHUNK, 1, K), jnp.int32),
        pltpu.VMEM((NCHUNK, 1, K), jnp.int32),
        pltpu.VMEM((K, C), jnp.float32),
        pltpu.VMEM((K, C), jnp.float32),
        pltpu.VMEM((40, C), jnp.float32),
        pltpu.VMEM_SHARED((NPAD, C), jnp.float32),
        pltpu.SemaphoreType.DMA,
        pltpu.SemaphoreType.DMA,
    ],
)
def _sc_push(t0, t1, t2, t3, t4, srcs, dsts, o0, o1, o2, o3, o4,
             sidx, didx, b0, b1, zbuf, acc, g0, g1):
    c = lax.axis_index("c")
    s = lax.axis_index("s")
    wid = c * 16 + s
    bufs = (b0, b1)
    sems = (g0, g1)

    pltpu.sync_copy(srcs.at[wid], sidx)
    pltpu.sync_copy(dsts.at[wid], didx)

    z16 = jnp.zeros((16,), jnp.float32)

    def _zrow(i, carry):
        for k in range(C // 16):
            zbuf[i, pl.ds(k * 16, 16)] = z16
        return carry

    lax.fori_loop(0, 40, _zrow, 0)

    for tab, out in ((t0, o0), (t1, o1), (t2, o2), (t3, o3), (t4, o4)):
        for z in range(RPT // 40):
            pltpu.sync_copy(zbuf, acc.at[pl.ds(s * RPT + z * 40, 40)])
        plsc.subcore_barrier()

        pltpu.async_copy(tab.at[sidx.at[0, 0]], bufs[0], sems[0])

        def _super(g, carry, tab=tab):
            for b in range(2):
                jj = 2 * g + b
                nxt = jj + 1

                @pl.when(nxt < NCHUNK)
                def _():
                    pltpu.async_copy(tab.at[sidx.at[nxt, 0]],
                                     bufs[1 - b], sems[1 - b])

                pltpu.make_async_copy(tab.at[sidx.at[jj, 0]],
                                      bufs[b], sems[b]).wait()
                pltpu.sync_copy(bufs[b], acc.at[didx.at[jj, 0]], add=True)
            return carry

        lax.fori_loop(0, NCHUNK // 2, _super, 0)
        plsc.subcore_barrier()

        pltpu.sync_copy(acc.at[pl.ds(s * RPT, RPT)],
                        out.at[c, pl.ds(s * RPT, RPT)])


def _combine(p_ref):
    a = p_ref[...]
    return a[0] + a[1]


def _aggr(xr_ref, p0_ref, p1_ref, p2_ref, p3_ref, p4_ref, sc_ref, bi_ref):
    s0 = _combine(p0_ref)
    s1 = _combine(p1_ref)
    s2 = _combine(p2_ref)
    s3 = _combine(p3_ref)
    s4 = _combine(p4_ref)
    den0 = s4[:, 0:1] + 1e-16
    den1 = s4[:, 1:2] + 1e-16
    a_lo = s0 / den0 + s2 / den1
    a_hi = s1 / den0 + s3 / den1
    h = xr_ref[...] + jnp.concatenate([a_lo, a_hi], axis=1)
    return jnp.maximum(h * sc_ref[...] + bi_ref[...], 0.0)


def _post_body(xr_ref, p0_ref, p1_ref, p2_ref, p3_ref, p4_ref,
               sc_ref, bi_ref, o_ref):
    o_ref[...] = _aggr(xr_ref, p0_ref, p1_ref, p2_ref, p3_ref, p4_ref,
                       sc_ref, bi_ref)


def _post_head_body(xr_ref, p0_ref, p1_ref, p2_ref, p3_ref, p4_ref,
                    sc_ref, bi_ref, hwt_ref, hb_ref, o_ref):
    h = _aggr(xr_ref, p0_ref, p1_ref, p2_ref, p3_ref, p4_ref, sc_ref, bi_ref)
    o_ref[...] = (jnp.dot(h, hwt_ref[...], preferred_element_type=jnp.float32)
                  + hb_ref[...])


_P_SPECS = [
    pl.BlockSpec((BN, D), lambda i: (i, 0)),
    pl.BlockSpec((2, BN, C), lambda i: (0, i, 0)),
    pl.BlockSpec((2, BN, C), lambda i: (0, i, 0)),
    pl.BlockSpec((2, BN, C), lambda i: (0, i, 0)),
    pl.BlockSpec((2, BN, C), lambda i: (0, i, 0)),
    pl.BlockSpec((2, BN, C), lambda i: (0, i, 0)),
    pl.BlockSpec((1, D), lambda i: (0, 0)),
    pl.BlockSpec((1, D), lambda i: (0, 0)),
]


def _mid_body(xr_ref, p0_ref, p1_ref, p2_ref, p3_ref, p4_ref, sc_ref, bi_ref,
              wrT_ref, waT_ref, wnT_ref, avq_ref, avm_ref,
              xr2_ref, t0_ref, t1_ref, t2_ref, t3_ref, t4_ref):
    h = _aggr(xr_ref, p0_ref, p1_ref, p2_ref, p3_ref, p4_ref, sc_ref, bi_ref)
    _dense_tables(h, wrT_ref, waT_ref, wnT_ref, avq_ref, avm_ref,
                  xr2_ref, t0_ref, t1_ref, t2_ref, t3_ref, t4_ref)


def _mid(xr, p0, p1, p2, p3, p4, sc, bi, wrT, waT, wnT, avq, avm):
    return pl.pallas_call(
        _mid_body,
        grid=(GRID,),
        in_specs=_P_SPECS + [
            pl.BlockSpec((D, D), lambda i: (0, 0)),
            pl.BlockSpec((D, 2 * D), lambda i: (0, 0)),
            pl.BlockSpec((D, 2 * D), lambda i: (0, 0)),
            pl.BlockSpec((1, 2 * D), lambda i: (0, 0)),
            pl.BlockSpec((1, 2 * D), lambda i: (0, 0)),
        ],
        out_specs=[pl.BlockSpec((BN, D), lambda i: (i, 0))]
        + [pl.BlockSpec((BN, C), lambda i: (i, 0))] * 5,
        out_shape=[jax.ShapeDtypeStruct((N, D), jnp.float32)]
        + [jax.ShapeDtypeStruct((N, C), jnp.float32)] * 5,
    )(xr, p0, p1, p2, p3, p4, sc, bi, wrT, waT, wnT, avq, avm)


def _post(xr, p0, p1, p2, p3, p4, sc, bi):
    return pl.pallas_call(
        _post_body,
        grid=(GRID,),
        in_specs=_P_SPECS,
        out_specs=pl.BlockSpec((BN, D), lambda i: (i, 0)),
        out_shape=jax.ShapeDtypeStruct((N, D), jnp.float32),
    )(xr, p0, p1, p2, p3, p4, sc, bi)


def _post_head(xr, p0, p1, p2, p3, p4, sc, bi, hwt, hb):
    return pl.pallas_call(
        _post_head_body,
        grid=(GRID,),
        in_specs=_P_SPECS + [
            pl.BlockSpec((D, 128), lambda i: (0, 0)),
            pl.BlockSpec((1, 128), lambda i: (0, 0)),
        ],
        out_specs=pl.BlockSpec((BN, 128), lambda i: (i, 0)),
        out_shape=jax.ShapeDtypeStruct((N, 128), jnp.float32),
    )(xr, p0, p1, p2, p3, p4, sc, bi, hwt, hb)


def kernel(x, edge_index, W_root0, W_neigh0, W_att0, att_vec0,
           bn_g0, bn_b0, bn_m0, bn_v0,
           W_root1, W_neigh1, W_att1, att_vec1,
           bn_g1, bn_b1, bn_m1, bn_v1, head_W, head_b):
    srcs = edge_index[0].reshape(NT, NCHUNK, 1, K)
    dsts = edge_index[1].reshape(NT, NCHUNK, 1, K)

    avq0 = att_vec0[:, :D].reshape(1, 2 * D)
    avm0 = att_vec0[:, D:].reshape(1, 2 * D)
    avq1 = att_vec1[:, :D].reshape(1, 2 * D)
    avm1 = att_vec1[:, D:].reshape(1, 2 * D)

    sc0 = (bn_g0 / jnp.sqrt(bn_v0 + 1e-5)).reshape(1, D)
    bi0 = bn_b0.reshape(1, D) - bn_m0.reshape(1, D) * sc0
    sc1 = (bn_g1 / jnp.sqrt(bn_v1 + 1e-5)).reshape(1, D)
    bi1 = bn_b1.reshape(1, D) - bn_m1.reshape(1, D) * sc1

    hwt = jnp.zeros((D, 128), jnp.float32).at[:, :3].set(head_W.T)
    hb = jnp.zeros((1, 128), jnp.float32).at[0, :3].set(head_b)

    xr1, t10, t11, t12, t13, t14 = _attn_dense(
        x, W_root0.T, W_att0.T, W_neigh0.T, avq0, avm0)
    q1 = _sc_push(t10, t11, t12, t13, t14, srcs, dsts)
    xr2, t20, t21, t22, t23, t24 = _mid(
        xr1, *q1, sc0, bi0, W_root1.T, W_att1.T, W_neigh1.T, avq1, avm1)
    q2 = _sc_push(t20, t21, t22, t23, t24, srcs, dsts)
    out = _post_head(xr2, *q2, sc1, bi1, hwt, hb)
    return out[:, :3]


# speedup vs baseline: 15.3098x; 1.0428x over previous
import functools

import jax
import jax.numpy as jnp
from jax import lax
from jax.experimental import pallas as pl
from jax.experimental.pallas import tpu as pltpu
from jax.experimental.pallas import tpu_sc as plsc

N = 10000
NPAD = 10240
E = 160000
D = 256
C = 128
BN = 400
GRID = N // BN
NT = 32
EPT = E // NT
K = 125
NCHUNK = EPT // K
RPT = NPAD // 16


def _dense_tables(xb, wrT_ref, waT_ref, wnT_ref, avq_ref, avm_ref,
                  xr_ref, t0_ref, t1_ref, t2_ref, t3_ref, t4_ref):
    xr_ref[...] = jnp.dot(xb, wrT_ref[...], preferred_element_type=jnp.float32)
    xq = jnp.dot(xb, waT_ref[...], preferred_element_type=jnp.float32)
    xm = jnp.dot(xb, wnT_ref[...], preferred_element_type=jnp.float32)
    s = (jnp.where(xq >= 0, xq, 0.2 * xq) * avq_ref[...]
         + jnp.where(xm >= 0, xm, 0.2 * xm) * avm_ref[...])
    w0 = jnp.exp(jnp.sum(s[:, :D], axis=1, keepdims=True))
    w1 = jnp.exp(jnp.sum(s[:, D:], axis=1, keepdims=True))
    y0 = xm[:, :D] * w0
    y1 = xm[:, D:] * w1
    t0_ref[...] = y0[:, :128]
    t1_ref[...] = y0[:, 128:]
    t2_ref[...] = y1[:, :128]
    t3_ref[...] = y1[:, 128:]
    t4_ref[...] = jnp.concatenate(
        [w0, w1, jnp.zeros((BN, C - 2), jnp.float32)], axis=1)


def _att_body(x_ref, wrT_ref, waT_ref, wnT_ref, avq_ref, avm_ref,
              xr_ref, t0_ref, t1_ref, t2_ref, t3_ref, t4_ref):
    _dense_tables(x_ref[...], wrT_ref, waT_ref, wnT_ref, avq_ref, avm_ref,
                  xr_ref, t0_ref, t1_ref, t2_ref, t3_ref, t4_ref)


def _attn_dense(x, wrT, waT, wnT, avq, avm):
    return pl.pallas_call(
        _att_body,
        grid=(GRID,),
        in_specs=[
            pl.BlockSpec((BN, D), lambda i: (i, 0)),
            pl.BlockSpec((D, D), lambda i: (0, 0)),
            pl.BlockSpec((D, 2 * D), lambda i: (0, 0)),
            pl.BlockSpec((D, 2 * D), lambda i: (0, 0)),
            pl.BlockSpec((1, 2 * D), lambda i: (0, 0)),
            pl.BlockSpec((1, 2 * D), lambda i: (0, 0)),
        ],
        out_specs=[pl.BlockSpec((BN, D), lambda i: (i, 0))]
        + [pl.BlockSpec((BN, C), lambda i: (i, 0))] * 5,
        out_shape=[jax.ShapeDtypeStruct((N, D), jnp.float32)]
        + [jax.ShapeDtypeStruct((N, C), jnp.float32)] * 5,
    )(x, wrT, waT, wnT, avq, avm)


_SC_MESH = plsc.VectorSubcoreMesh(core_axis_name="c", subcore_axis_name="s")


@functools.partial(
    pl.kernel,
    out_type=[jax.ShapeDtypeStruct((2, NPAD, C), jnp.float32)] * 5,
    mesh=_SC_MESH,
    scratch_types=[
        pltpu.VMEM((NCHUNK, 1, K), jnp.int32),
        pltpu.VMEM((NCHUNK, 1, K), jnp.int32),
        pltpu.VMEM((K, C), jnp.float32),
        pltpu.VMEM((K, C), jnp.float32),
        pltpu.VMEM((40, C), jnp.float32),
        pltpu.VMEM_SHARED((NPAD, C), jnp.float32),
        pltpu.SemaphoreType.DMA,
        pltpu.SemaphoreType.DMA,
    ],
)
def _sc_push(t0, t1, t2, t3, t4, srcs, dsts, o0, o1, o2, o3, o4,
             sidx, didx, b0, b1, zbuf, acc, g0, g1):
    c = lax.axis_index("c")
    s = lax.axis_index("s")
    wid = c * 16 + s
    bufs = (b0, b1)
    sems = (g0, g1)

    pltpu.sync_copy(srcs.at[wid], sidx)
    pltpu.sync_copy(dsts.at[wid], didx)

    z16 = jnp.zeros((16,), jnp.float32)

    def _zrow(i, carry):
        for k in range(C // 16):
            zbuf[i, pl.ds(k * 16, 16)] = z16
        return carry

    lax.fori_loop(0, 40, _zrow, 0)

    for z in range(RPT // 40):
        pltpu.sync_copy(zbuf, acc.at[pl.ds(s * RPT + z * 40, 40)])
    plsc.subcore_barrier()

    for tab, out in ((t0, o0), (t1, o1), (t2, o2), (t3, o3), (t4, o4)):
        pltpu.async_copy(tab.at[sidx.at[0, 0]], bufs[0], sems[0])

        def _super(g, carry, tab=tab):
            for b in range(2):
                jj = 2 * g + b
                nxt = jj + 1

                @pl.when(nxt < NCHUNK)
                def _():
                    pltpu.async_copy(tab.at[sidx.at[nxt, 0]],
                                     bufs[1 - b], sems[1 - b])

                pltpu.make_async_copy(tab.at[sidx.at[jj, 0]],
                                      bufs[b], sems[b]).wait()
                pltpu.sync_copy(bufs[b], acc.at[didx.at[jj, 0]], add=True)
            return carry

        lax.fori_loop(0, NCHUNK // 2, _super, 0)
        plsc.subcore_barrier()

        pltpu.sync_copy(acc.at[pl.ds(s * RPT, RPT)],
                        out.at[c, pl.ds(s * RPT, RPT)])
        plsc.subcore_barrier()


def _combine(p_ref):
    a = p_ref[...]
    return a[0] + a[1]


def _aggr(xr_ref, p0_ref, p1_ref, p2_ref, p3_ref, p4_ref, sc_ref, bi_ref):
    d0 = _combine(p0_ref)
    d1 = _combine(p1_ref)
    d2 = _combine(p2_ref)
    d3 = _combine(p3_ref)
    d4 = _combine(p4_ref)
    s0 = d0
    s1 = d1 - d0
    s2 = d2 - d1
    s3 = d3 - d2
    s4 = d4 - d3
    den0 = s4[:, 0:1] + 1e-16
    den1 = s4[:, 1:2] + 1e-16
    a_lo = s0 / den0 + s2 / den1
    a_hi = s1 / den0 + s3 / den1
    h = xr_ref[...] + jnp.concatenate([a_lo, a_hi], axis=1)
    return jnp.maximum(h * sc_ref[...] + bi_ref[...], 0.0)


def _post_body(xr_ref, p0_ref, p1_ref, p2_ref, p3_ref, p4_ref,
               sc_ref, bi_ref, o_ref):
    o_ref[...] = _aggr(xr_ref, p0_ref, p1_ref, p2_ref, p3_ref, p4_ref,
                       sc_ref, bi_ref)


def _post_head_body(xr_ref, p0_ref, p1_ref, p2_ref, p3_ref, p4_ref,
                    sc_ref, bi_ref, hwt_ref, hb_ref, o_ref):
    h = _aggr(xr_ref, p0_ref, p1_ref, p2_ref, p3_ref, p4_ref, sc_ref, bi_ref)
    o_ref[...] = (jnp.dot(h, hwt_ref[...], preferred_element_type=jnp.float32)
                  + hb_ref[...])


_P_SPECS = [
    pl.BlockSpec((BN, D), lambda i: (i, 0)),
    pl.BlockSpec((2, BN, C), lambda i: (0, i, 0)),
    pl.BlockSpec((2, BN, C), lambda i: (0, i, 0)),
    pl.BlockSpec((2, BN, C), lambda i: (0, i, 0)),
    pl.BlockSpec((2, BN, C), lambda i: (0, i, 0)),
    pl.BlockSpec((2, BN, C), lambda i: (0, i, 0)),
    pl.BlockSpec((1, D), lambda i: (0, 0)),
    pl.BlockSpec((1, D), lambda i: (0, 0)),
]


def _mid_body(xr_ref, p0_ref, p1_ref, p2_ref, p3_ref, p4_ref, sc_ref, bi_ref,
              wrT_ref, waT_ref, wnT_ref, avq_ref, avm_ref,
              xr2_ref, t0_ref, t1_ref, t2_ref, t3_ref, t4_ref):
    h = _aggr(xr_ref, p0_ref, p1_ref, p2_ref, p3_ref, p4_ref, sc_ref, bi_ref)
    _dense_tables(h, wrT_ref, waT_ref, wnT_ref, avq_ref, avm_ref,
                  xr2_ref, t0_ref, t1_ref, t2_ref, t3_ref, t4_ref)


def _mid(xr, p0, p1, p2, p3, p4, sc, bi, wrT, waT, wnT, avq, avm):
    return pl.pallas_call(
        _mid_body,
        grid=(GRID,),
        in_specs=_P_SPECS + [
            pl.BlockSpec((D, D), lambda i: (0, 0)),
            pl.BlockSpec((D, 2 * D), lambda i: (0, 0)),
            pl.BlockSpec((D, 2 * D), lambda i: (0, 0)),
            pl.BlockSpec((1, 2 * D), lambda i: (0, 0)),
            pl.BlockSpec((1, 2 * D), lambda i: (0, 0)),
        ],
        out_specs=[pl.BlockSpec((BN, D), lambda i: (i, 0))]
        + [pl.BlockSpec((BN, C), lambda i: (i, 0))] * 5,
        out_shape=[jax.ShapeDtypeStruct((N, D), jnp.float32)]
        + [jax.ShapeDtypeStruct((N, C), jnp.float32)] * 5,
    )(xr, p0, p1, p2, p3, p4, sc, bi, wrT, waT, wnT, avq, avm)


def _post(xr, p0, p1, p2, p3, p4, sc, bi):
    return pl.pallas_call(
        _post_body,
        grid=(GRID,),
        in_specs=_P_SPECS,
        out_specs=pl.BlockSpec((BN, D), lambda i: (i, 0)),
        out_shape=jax.ShapeDtypeStruct((N, D), jnp.float32),
    )(xr, p0, p1, p2, p3, p4, sc, bi)


def _post_head(xr, p0, p1, p2, p3, p4, sc, bi, hwt, hb):
    return pl.pallas_call(
        _post_head_body,
        grid=(GRID,),
        in_specs=_P_SPECS + [
            pl.BlockSpec((D, 128), lambda i: (0, 0)),
            pl.BlockSpec((1, 128), lambda i: (0, 0)),
        ],
        out_specs=pl.BlockSpec((BN, 128), lambda i: (i, 0)),
        out_shape=jax.ShapeDtypeStruct((N, 128), jnp.float32),
    )(xr, p0, p1, p2, p3, p4, sc, bi, hwt, hb)


def kernel(x, edge_index, W_root0, W_neigh0, W_att0, att_vec0,
           bn_g0, bn_b0, bn_m0, bn_v0,
           W_root1, W_neigh1, W_att1, att_vec1,
           bn_g1, bn_b1, bn_m1, bn_v1, head_W, head_b):
    srcs = edge_index[0].reshape(NT, NCHUNK, 1, K)
    dsts = edge_index[1].reshape(NT, NCHUNK, 1, K)

    avq0 = att_vec0[:, :D].reshape(1, 2 * D)
    avm0 = att_vec0[:, D:].reshape(1, 2 * D)
    avq1 = att_vec1[:, :D].reshape(1, 2 * D)
    avm1 = att_vec1[:, D:].reshape(1, 2 * D)

    sc0 = (bn_g0 / jnp.sqrt(bn_v0 + 1e-5)).reshape(1, D)
    bi0 = bn_b0.reshape(1, D) - bn_m0.reshape(1, D) * sc0
    sc1 = (bn_g1 / jnp.sqrt(bn_v1 + 1e-5)).reshape(1, D)
    bi1 = bn_b1.reshape(1, D) - bn_m1.reshape(1, D) * sc1

    hwt = jnp.zeros((D, 128), jnp.float32).at[:, :3].set(head_W.T)
    hb = jnp.zeros((1, 128), jnp.float32).at[0, :3].set(head_b)

    xr1, t10, t11, t12, t13, t14 = _attn_dense(
        x, W_root0.T, W_att0.T, W_neigh0.T, avq0, avm0)
    q1 = _sc_push(t10, t11, t12, t13, t14, srcs, dsts)
    xr2, t20, t21, t22, t23, t24 = _mid(
        xr1, *q1, sc0, bi0, W_root1.T, W_att1.T, W_neigh1.T, avq1, avm1)
    q2 = _sc_push(t20, t21, t22, t23, t24, srcs, dsts)
    out = _post_head(xr2, *q2, sc1, bi1, hwt, hb)
    return out[:, :3]


# speedup vs baseline: 15.5887x; 1.0182x over previous
import functools

import jax
import jax.numpy as jnp
from jax import lax
from jax.experimental import pallas as pl
from jax.experimental.pallas import tpu as pltpu
from jax.experimental.pallas import tpu_sc as plsc

N = 10000
NPAD = 10240
E = 160000
D = 256
C = 128
BN = 400
GRID = N // BN
NT = 32
EPT = E // NT
K = 125
NCHUNK = EPT // K
RPT = NPAD // 16


def _dense_tables(xb, wrT_ref, waT_ref, wnT_ref, avq_ref, avm_ref,
                  xr_ref, t0_ref, t1_ref, t2_ref, t3_ref, t4_ref):
    xr_ref[...] = jnp.dot(xb, wrT_ref[...], preferred_element_type=jnp.float32)
    xq = jnp.dot(xb, waT_ref[...], preferred_element_type=jnp.float32)
    xm = jnp.dot(xb, wnT_ref[...], preferred_element_type=jnp.float32)
    s = (jnp.where(xq >= 0, xq, 0.2 * xq) * avq_ref[...]
         + jnp.where(xm >= 0, xm, 0.2 * xm) * avm_ref[...])
    w0 = jnp.exp(jnp.sum(s[:, :D], axis=1, keepdims=True))
    w1 = jnp.exp(jnp.sum(s[:, D:], axis=1, keepdims=True))
    y0 = xm[:, :D] * w0
    y1 = xm[:, D:] * w1
    t0_ref[...] = y0[:, :128]
    t1_ref[...] = y0[:, 128:]
    t2_ref[...] = y1[:, :128]
    t3_ref[...] = y1[:, 128:]
    t4_ref[...] = jnp.concatenate(
        [w0, w1, jnp.zeros((BN, C - 2), jnp.float32)], axis=1)


def _att_body(x_ref, wrT_ref, waT_ref, wnT_ref, avq_ref, avm_ref,
              xr_ref, t0_ref, t1_ref, t2_ref, t3_ref, t4_ref):
    _dense_tables(x_ref[...], wrT_ref, waT_ref, wnT_ref, avq_ref, avm_ref,
                  xr_ref, t0_ref, t1_ref, t2_ref, t3_ref, t4_ref)


def _attn_dense(x, wrT, waT, wnT, avq, avm):
    return pl.pallas_call(
        _att_body,
        grid=(GRID,),
        in_specs=[
            pl.BlockSpec((BN, D), lambda i: (i, 0)),
            pl.BlockSpec((D, D), lambda i: (0, 0)),
            pl.BlockSpec((D, 2 * D), lambda i: (0, 0)),
            pl.BlockSpec((D, 2 * D), lambda i: (0, 0)),
            pl.BlockSpec((1, 2 * D), lambda i: (0, 0)),
            pl.BlockSpec((1, 2 * D), lambda i: (0, 0)),
        ],
        out_specs=[pl.BlockSpec((BN, D), lambda i: (i, 0))]
        + [pl.BlockSpec((BN, C), lambda i: (i, 0))] * 5,
        out_shape=[jax.ShapeDtypeStruct((N, D), jnp.float32)]
        + [jax.ShapeDtypeStruct((N, C), jnp.float32)] * 5,
    )(x, wrT, waT, wnT, avq, avm)


_SC_MESH = plsc.VectorSubcoreMesh(core_axis_name="c", subcore_axis_name="s")


@functools.partial(
    pl.kernel,
    out_type=[jax.ShapeDtypeStruct((2, NPAD, C), jnp.float32)] * 5,
    mesh=_SC_MESH,
    scratch_types=[
        pltpu.VMEM((NCHUNK, 1, K), jnp.int32),
        pltpu.VMEM((NCHUNK, 1, K), jnp.int32),
        pltpu.VMEM((K, C), jnp.float32),
        pltpu.VMEM((K, C), jnp.float32),
        pltpu.VMEM((40, C), jnp.float32),
        pltpu.VMEM_SHARED((NPAD, C), jnp.float32),
        pltpu.SemaphoreType.DMA,
        pltpu.SemaphoreType.DMA,
    ],
)
def _sc_push(t0, t1, t2, t3, t4, srcs, dsts, o0, o1, o2, o3, o4,
             sidx, didx, b0, b1, zbuf, acc, g0, g1):
    c = lax.axis_index("c")
    s = lax.axis_index("s")
    wid = c * 16 + s
    bufs = (b0, b1)
    sems = (g0, g1)

    pltpu.sync_copy(srcs.at[wid], sidx)
    pltpu.sync_copy(dsts.at[wid], didx)

    z16 = jnp.zeros((16,), jnp.float32)

    def _zrow(i, carry):
        for k in range(C // 16):
            zbuf[i, pl.ds(k * 16, 16)] = z16
        return carry

    lax.fori_loop(0, 40, _zrow, 0)

    for z in range(RPT // 40):
        pltpu.sync_copy(zbuf, acc.at[pl.ds(s * RPT + z * 40, 40)])
    plsc.subcore_barrier()

    tabs = ((t0, o0), (t1, o1), (t2, o2), (t3, o3), (t4, o4))
    pltpu.async_copy(t0.at[sidx.at[0, 0]], bufs[0], sems[0])

    for p, (tab, out) in enumerate(tabs):
        def _super(g, carry, tab=tab):
            for b in range(2):
                jj = 2 * g + b
                nxt = jj + 1

                @pl.when(nxt < NCHUNK)
                def _():
                    pltpu.async_copy(tab.at[sidx.at[nxt, 0]],
                                     bufs[1 - b], sems[1 - b])

                pltpu.make_async_copy(tab.at[sidx.at[jj, 0]],
                                      bufs[b], sems[b]).wait()
                pltpu.sync_copy(bufs[b], acc.at[didx.at[jj, 0]], add=True)
            return carry

        lax.fori_loop(0, NCHUNK // 2, _super, 0)
        plsc.subcore_barrier()

        if p + 1 < len(tabs):
            pltpu.async_copy(tabs[p + 1][0].at[sidx.at[0, 0]],
                             bufs[0], sems[0])
        pltpu.sync_copy(acc.at[pl.ds(s * RPT, RPT)],
                        out.at[c, pl.ds(s * RPT, RPT)])
        plsc.subcore_barrier()


def _combine(p_ref):
    a = p_ref[...]
    return a[0] + a[1]


def _aggr(xr_ref, p0_ref, p1_ref, p2_ref, p3_ref, p4_ref, sc_ref, bi_ref):
    d0 = _combine(p0_ref)
    d1 = _combine(p1_ref)
    d2 = _combine(p2_ref)
    d3 = _combine(p3_ref)
    d4 = _combine(p4_ref)
    s0 = d0
    s1 = d1 - d0
    s2 = d2 - d1
    s3 = d3 - d2
    s4 = d4 - d3
    den0 = s4[:, 0:1] + 1e-16
    den1 = s4[:, 1:2] + 1e-16
    a_lo = s0 / den0 + s2 / den1
    a_hi = s1 / den0 + s3 / den1
    h = xr_ref[...] + jnp.concatenate([a_lo, a_hi], axis=1)
    return jnp.maximum(h * sc_ref[...] + bi_ref[...], 0.0)


def _post_body(xr_ref, p0_ref, p1_ref, p2_ref, p3_ref, p4_ref,
               sc_ref, bi_ref, o_ref):
    o_ref[...] = _aggr(xr_ref, p0_ref, p1_ref, p2_ref, p3_ref, p4_ref,
                       sc_ref, bi_ref)


def _post_head_body(xr_ref, p0_ref, p1_ref, p2_ref, p3_ref, p4_ref,
                    sc_ref, bi_ref, hwt_ref, hb_ref, o_ref):
    h = _aggr(xr_ref, p0_ref, p1_ref, p2_ref, p3_ref, p4_ref, sc_ref, bi_ref)
    o_ref[...] = (jnp.dot(h, hwt_ref[...], preferred_element_type=jnp.float32)
                  + hb_ref[...])


_P_SPECS = [
    pl.BlockSpec((BN, D), lambda i: (i, 0)),
    pl.BlockSpec((2, BN, C), lambda i: (0, i, 0)),
    pl.BlockSpec((2, BN, C), lambda i: (0, i, 0)),
    pl.BlockSpec((2, BN, C), lambda i: (0, i, 0)),
    pl.BlockSpec((2, BN, C), lambda i: (0, i, 0)),
    pl.BlockSpec((2, BN, C), lambda i: (0, i, 0)),
    pl.BlockSpec((1, D), lambda i: (0, 0)),
    pl.BlockSpec((1, D), lambda i: (0, 0)),
]


def _mid_body(xr_ref, p0_ref, p1_ref, p2_ref, p3_ref, p4_ref, sc_ref, bi_ref,
              wrT_ref, waT_ref, wnT_ref, avq_ref, avm_ref,
              xr2_ref, t0_ref, t1_ref, t2_ref, t3_ref, t4_ref):
    h = _aggr(xr_ref, p0_ref, p1_ref, p2_ref, p3_ref, p4_ref, sc_ref, bi_ref)
    _dense_tables(h, wrT_ref, waT_ref, wnT_ref, avq_ref, avm_ref,
                  xr2_ref, t0_ref, t1_ref, t2_ref, t3_ref, t4_ref)


def _mid(xr, p0, p1, p2, p3, p4, sc, bi, wrT, waT, wnT, avq, avm):
    return pl.pallas_call(
        _mid_body,
        grid=(GRID,),
        in_specs=_P_SPECS + [
            pl.BlockSpec((D, D), lambda i: (0, 0)),
            pl.BlockSpec((D, 2 * D), lambda i: (0, 0)),
            pl.BlockSpec((D, 2 * D), lambda i: (0, 0)),
            pl.BlockSpec((1, 2 * D), lambda i: (0, 0)),
            pl.BlockSpec((1, 2 * D), lambda i: (0, 0)),
        ],
        out_specs=[pl.BlockSpec((BN, D), lambda i: (i, 0))]
        + [pl.BlockSpec((BN, C), lambda i: (i, 0))] * 5,
        out_shape=[jax.ShapeDtypeStruct((N, D), jnp.float32)]
        + [jax.ShapeDtypeStruct((N, C), jnp.float32)] * 5,
    )(xr, p0, p1, p2, p3, p4, sc, bi, wrT, waT, wnT, avq, avm)


def _post(xr, p0, p1, p2, p3, p4, sc, bi):
    return pl.pallas_call(
        _post_body,
        grid=(GRID,),
        in_specs=_P_SPECS,
        out_specs=pl.BlockSpec((BN, D), lambda i: (i, 0)),
        out_shape=jax.ShapeDtypeStruct((N, D), jnp.float32),
    )(xr, p0, p1, p2, p3, p4, sc, bi)


def _post_head(xr, p0, p1, p2, p3, p4, sc, bi, hwt, hb):
    return pl.pallas_call(
        _post_head_body,
        grid=(GRID,),
        in_specs=_P_SPECS + [
            pl.BlockSpec((D, 128), lambda i: (0, 0)),
            pl.BlockSpec((1, 128), lambda i: (0, 0)),
        ],
        out_specs=pl.BlockSpec((BN, 128), lambda i: (i, 0)),
        out_shape=jax.ShapeDtypeStruct((N, 128), jnp.float32),
    )(xr, p0, p1, p2, p3, p4, sc, bi, hwt, hb)


def kernel(x, edge_index, W_root0, W_neigh0, W_att0, att_vec0,
           bn_g0, bn_b0, bn_m0, bn_v0,
           W_root1, W_neigh1, W_att1, att_vec1,
           bn_g1, bn_b1, bn_m1, bn_v1, head_W, head_b):
    srcs = edge_index[0].reshape(NT, NCHUNK, 1, K)
    dsts = edge_index[1].reshape(NT, NCHUNK, 1, K)

    avq0 = att_vec0[:, :D].reshape(1, 2 * D)
    avm0 = att_vec0[:, D:].reshape(1, 2 * D)
    avq1 = att_vec1[:, :D].reshape(1, 2 * D)
    avm1 = att_vec1[:, D:].reshape(1, 2 * D)

    sc0 = (bn_g0 / jnp.sqrt(bn_v0 + 1e-5)).reshape(1, D)
    bi0 = bn_b0.reshape(1, D) - bn_m0.reshape(1, D) * sc0
    sc1 = (bn_g1 / jnp.sqrt(bn_v1 + 1e-5)).reshape(1, D)
    bi1 = bn_b1.reshape(1, D) - bn_m1.reshape(1, D) * sc1

    hwt = jnp.zeros((D, 128), jnp.float32).at[:, :3].set(head_W.T)
    hb = jnp.zeros((1, 128), jnp.float32).at[0, :3].set(head_b)

    xr1, t10, t11, t12, t13, t14 = _attn_dense(
        x, W_root0.T, W_att0.T, W_neigh0.T, avq0, avm0)
    q1 = _sc_push(t10, t11, t12, t13, t14, srcs, dsts)
    xr2, t20, t21, t22, t23, t24 = _mid(
        xr1, *q1, sc0, bi0, W_root1.T, W_att1.T, W_neigh1.T, avq1, avm1)
    q2 = _sc_push(t20, t21, t22, t23, t24, srcs, dsts)
    out = _post_head(xr2, *q2, sc1, bi1, hwt, hb)
    return out[:, :3]
